# baseline (speedup 1.0000x reference)
import functools
import math

import jax
import jax.numpy as jnp
from jax.experimental import pallas as pl
from jax.experimental.pallas import tpu as pltpu


def _se_packed_body(x_ref, seg_ref, segt_ref, w1_ref, b1_ref, w2_ref,
                    b2_ref, o_ref, *, mean_scale, groups):
    xs = x_ref[...]

    pooled = jax.lax.dot_general(
        xs, seg_ref[...], (((2,), (0,)), ((), ())),
        preferred_element_type=jnp.float32) * mean_scale

    pooled = pooled.reshape(pooled.shape[0], -1)
    hid = jax.lax.dot_general(
        pooled, w1_ref[...], (((1,), (0,)), ((), ())),
        preferred_element_type=jnp.float32)
    hid = jnp.maximum(hid + b1_ref[...], 0.0)

    act = jax.lax.dot_general(
        hid, w2_ref[...], (((1,), (0,)), ((), ())),
        preferred_element_type=jnp.float32) + b2_ref[...]
    gate = jax.nn.sigmoid(act)

    gate_rows = gate.reshape(gate.shape[0], groups, -1)
    gate_flat = jax.lax.dot_general(
        gate_rows, segt_ref[...], (((2,), (0,)), ((), ())),
        preferred_element_type=jnp.float32)
    o_ref[...] = xs * gate_flat.astype(o_ref.dtype)


def _se_simple_body(x_ref, w1_ref, b1_ref, w2t_ref, b2_ref, o_ref, *,
                    mean_scale):
    xs = x_ref[...]
    col = jnp.sum(xs, axis=2, keepdims=True, dtype=jnp.float32) * mean_scale
    hid = jnp.sum(col * w1_ref[...][None], axis=1, keepdims=True)
    hid = jnp.maximum(hid + b1_ref[...], 0.0)
    act = jnp.sum(hid * w2t_ref[...][None], axis=2, keepdims=True)
    gate = jax.nn.sigmoid(act + b2_ref[...][None])
    o_ref[...] = xs * gate.astype(o_ref.dtype)


def _pick_tile(B, per_image_bytes, target_bytes=4 * 1024 * 1024):
    tile = 1
    for cand in range(2, B + 1):
        if B % cand:
            continue
        if cand * per_image_bytes > target_bytes or B // cand < 4:
            break
        tile = cand
    return tile


@jax.jit
def kernel(x, w1, b1, w2, b2):
    B, C, H, W = x.shape
    Cr = w1.shape[1]
    S = H * W

    tile = _pick_tile(B, C * S * x.dtype.itemsize)
    block_bytes = tile * C * S * x.dtype.itemsize
    vmem_limit = min(7 * block_bytes + 16 * 1024 * 1024, 120 * 1024 * 1024)

    K = 128 // math.gcd(S, 128)
    packable = (C % K == 0)

    if packable:
        R = C // K
        L = K * S
        xs = x.reshape(B, R, L)

        lane = jnp.arange(L, dtype=jnp.int32)[:, None]
        chan = jnp.arange(K, dtype=jnp.int32)[None, :]
        seg = (lane // S == chan).astype(jnp.float32)

        body = functools.partial(_se_packed_body, mean_scale=1.0 / S,
                                 groups=R)
        out = pl.pallas_call(
            body,
            out_shape=jax.ShapeDtypeStruct((B, R, L), x.dtype),
            grid=(B // tile,),
            in_specs=[
                pl.BlockSpec((tile, R, L), lambda b: (b, 0, 0)),
                pl.BlockSpec((L, K), lambda b: (0, 0)),
                pl.BlockSpec((K, L), lambda b: (0, 0)),
                pl.BlockSpec((C, Cr), lambda b: (0, 0)),
                pl.BlockSpec((1, Cr), lambda b: (0, 0)),
                pl.BlockSpec((Cr, C), lambda b: (0, 0)),
                pl.BlockSpec((1, C), lambda b: (0, 0)),
            ],
            out_specs=pl.BlockSpec((tile, R, L), lambda b: (b, 0, 0)),
            compiler_params=pltpu.CompilerParams(
                dimension_semantics=("parallel",),
                vmem_limit_bytes=vmem_limit),
        )(xs, seg, seg.T, w1, b1.reshape(1, Cr), w2, b2.reshape(1, C))
    else:
        xs = x.reshape(B, C, S)
        body = functools.partial(_se_simple_body, mean_scale=1.0 / S)
        out = pl.pallas_call(
            body,
            out_shape=jax.ShapeDtypeStruct((B, C, S), x.dtype),
            grid=(B // tile,),
            in_specs=[
                pl.BlockSpec((tile, C, S), lambda b: (b, 0, 0)),
                pl.BlockSpec((C, Cr), lambda b: (0, 0)),
                pl.BlockSpec((1, 1, Cr), lambda b: (0, 0, 0)),
                pl.BlockSpec((C, Cr), lambda b: (0, 0)),
                pl.BlockSpec((C, 1), lambda b: (0, 0)),
            ],
            out_specs=pl.BlockSpec((tile, C, S), lambda b: (b, 0, 0)),
            compiler_params=pltpu.CompilerParams(
                dimension_semantics=("parallel",),
                vmem_limit_bytes=vmem_limit),
        )(xs, w1, b1.reshape(1, 1, Cr), w2.T, b2.reshape(C, 1))

    return out.reshape(B, C, H, W)


# speedup vs baseline: 12.8300x; 12.8300x over previous
import functools

import jax
import jax.numpy as jnp
from jax.experimental import pallas as pl
from jax.experimental.pallas import tpu as pltpu


def _se_body(x_ref, w1_ref, b1_ref, w2_ref, b2_ref, o_ref, *, mean_scale):
    xs = x_ref[...]

    pooled = jnp.sum(xs, axis=(0, 1), dtype=jnp.float32) * mean_scale

    hid = jax.lax.dot_general(
        pooled, w1_ref[...], (((1,), (0,)), ((), ())),
        preferred_element_type=jnp.float32)
    hid = jnp.maximum(hid + b1_ref[...], 0.0)

    act = jax.lax.dot_general(
        hid, w2_ref[...], (((1,), (0,)), ((), ())),
        preferred_element_type=jnp.float32) + b2_ref[...]
    gate = jax.nn.sigmoid(act)

    o_ref[...] = xs * gate[None, None].astype(o_ref.dtype)


def _pick_batch_tile(B, per_batch_bytes):
    tile = min(8, B)
    while B % tile:
        tile += 1
    while tile < B and tile * per_batch_bytes < 2 * 1024 * 1024:
        tile *= 2
        while B % tile:
            tile += 1
    return tile


@jax.jit
def kernel(x, w1, b1, w2, b2):
    B, C, H, W = x.shape
    Cr = w1.shape[1]

    xt = jnp.transpose(x, (2, 3, 0, 1))
    tile = _pick_batch_tile(B, C * H * W * x.dtype.itemsize)
    block_bytes = H * W * tile * C * x.dtype.itemsize
    vmem_limit = min(4 * block_bytes + 8 * 1024 * 1024, 112 * 1024 * 1024)

    body = functools.partial(_se_body, mean_scale=1.0 / (H * W))
    out_t = pl.pallas_call(
        body,
        out_shape=jax.ShapeDtypeStruct((H, W, B, C), x.dtype),
        grid=(B // tile,),
        in_specs=[
            pl.BlockSpec((H, W, tile, C), lambda b: (0, 0, b, 0)),
            pl.BlockSpec((C, Cr), lambda b: (0, 0)),
            pl.BlockSpec((1, Cr), lambda b: (0, 0)),
            pl.BlockSpec((Cr, C), lambda b: (0, 0)),
            pl.BlockSpec((1, C), lambda b: (0, 0)),
        ],
        out_specs=pl.BlockSpec((H, W, tile, C), lambda b: (0, 0, b, 0)),
        compiler_params=pltpu.CompilerParams(
            dimension_semantics=("parallel",),
            vmem_limit_bytes=vmem_limit),
    )(xt, w1, b1.reshape(1, Cr), w2, b2.reshape(1, C))

    return jnp.transpose(out_t, (2, 3, 0, 1))


# speedup vs baseline: 15.0844x; 1.1757x over previous
import functools

import jax
import jax.numpy as jnp
from jax.experimental import pallas as pl
from jax.experimental.pallas import tpu as pltpu


def _se_body(x_ref, w1_ref, b1_ref, w2_ref, b2_ref, o_ref, *, mean_scale):
    xs = x_ref[...]

    pooled = jnp.sum(xs, axis=(0, 1), dtype=jnp.float32) * mean_scale

    hid = jax.lax.dot_general(
        pooled, w1_ref[...], (((1,), (0,)), ((), ())),
        preferred_element_type=jnp.float32)
    hid = jnp.maximum(hid + b1_ref[...], 0.0)

    act = jax.lax.dot_general(
        hid, w2_ref[...], (((1,), (0,)), ((), ())),
        preferred_element_type=jnp.float32) + b2_ref[...]
    gate = jax.nn.sigmoid(act)

    o_ref[...] = xs * gate[None, None].astype(o_ref.dtype)


def _pick_batch_tile(B, per_batch_bytes):
    tile = min(16, B)
    while B % tile:
        tile += 1
    while tile < B and tile * per_batch_bytes < 2 * 1024 * 1024:
        tile *= 2
        while B % tile:
            tile += 1
    return tile


@jax.jit
def kernel(x, w1, b1, w2, b2):
    B, C, H, W = x.shape
    Cr = w1.shape[1]

    xt = jnp.transpose(x, (2, 3, 0, 1))
    tile = _pick_batch_tile(B, C * H * W * x.dtype.itemsize)
    block_bytes = H * W * tile * C * x.dtype.itemsize
    vmem_limit = min(4 * block_bytes + 8 * 1024 * 1024, 112 * 1024 * 1024)

    body = functools.partial(_se_body, mean_scale=1.0 / (H * W))
    out_t = pl.pallas_call(
        body,
        out_shape=jax.ShapeDtypeStruct((H, W, B, C), x.dtype),
        grid=(B // tile,),
        in_specs=[
            pl.BlockSpec((H, W, tile, C), lambda b: (0, 0, b, 0)),
            pl.BlockSpec((C, Cr), lambda b: (0, 0)),
            pl.BlockSpec((1, Cr), lambda b: (0, 0)),
            pl.BlockSpec((Cr, C), lambda b: (0, 0)),
            pl.BlockSpec((1, C), lambda b: (0, 0)),
        ],
        out_specs=pl.BlockSpec((H, W, tile, C), lambda b: (0, 0, b, 0)),
        compiler_params=pltpu.CompilerParams(
            dimension_semantics=("parallel",),
            vmem_limit_bytes=vmem_limit),
    )(xt, w1, b1.reshape(1, Cr), w2, b2.reshape(1, C))

    return jnp.transpose(out_t, (2, 3, 0, 1))


# speedup vs baseline: 15.8844x; 1.0530x over previous
import functools

import jax
import jax.numpy as jnp
from jax.experimental import pallas as pl
from jax.experimental.pallas import tpu as pltpu


def _se_body(x_ref, w1t_ref, b1_ref, w2_ref, b2_ref, o_ref, *, mean_scale):
    xs = x_ref[...]

    pooled = jnp.sum(xs, axis=(0, 1), dtype=jnp.float32) * mean_scale

    hid = jax.lax.dot_general(
        pooled, w1t_ref[...], (((1,), (1,)), ((), ())),
        preferred_element_type=jnp.float32)
    hid = jnp.maximum(hid + b1_ref[...], 0.0)

    act = jax.lax.dot_general(
        hid, w2_ref[...], (((1,), (0,)), ((), ())),
        preferred_element_type=jnp.float32) + b2_ref[...]
    gate = jax.nn.sigmoid(act)

    o_ref[...] = xs * gate[None, None].astype(o_ref.dtype)


def _pick_batch_tile(B, per_batch_bytes):
    tile = min(16, B)
    while B % tile:
        tile += 1
    while tile < B and tile * per_batch_bytes < 2 * 1024 * 1024:
        tile *= 2
        while B % tile:
            tile += 1
    return tile


@jax.jit
def kernel(x, w1, b1, w2, b2):
    B, C, H, W = x.shape
    Cr = w1.shape[1]

    xt = jnp.transpose(x, (2, 3, 0, 1))
    tile = _pick_batch_tile(B, C * H * W * x.dtype.itemsize)
    block_bytes = H * W * tile * C * x.dtype.itemsize
    vmem_limit = min(4 * block_bytes + 8 * 1024 * 1024, 112 * 1024 * 1024)

    body = functools.partial(_se_body, mean_scale=1.0 / (H * W))
    out_t = pl.pallas_call(
        body,
        out_shape=jax.ShapeDtypeStruct((H, W, B, C), x.dtype),
        grid=(B // tile,),
        in_specs=[
            pl.BlockSpec((H, W, tile, C), lambda b: (0, 0, b, 0)),
            pl.BlockSpec((Cr, C), lambda b: (0, 0)),
            pl.BlockSpec((1, Cr), lambda b: (0, 0)),
            pl.BlockSpec((Cr, C), lambda b: (0, 0)),
            pl.BlockSpec((1, C), lambda b: (0, 0)),
        ],
        out_specs=pl.BlockSpec((H, W, tile, C), lambda b: (0, 0, b, 0)),
        compiler_params=pltpu.CompilerParams(
            dimension_semantics=("parallel",),
            vmem_limit_bytes=vmem_limit),
    )(xt, w1.T, b1.reshape(1, Cr), w2, b2.reshape(1, C))

    return jnp.transpose(out_t, (2, 3, 0, 1))


# speedup vs baseline: 15.9023x; 1.0011x over previous
import functools

import jax
import jax.numpy as jnp
from jax.experimental import pallas as pl
from jax.experimental.pallas import tpu as pltpu


def _se_body(x_ref, w1t_ref, b1_ref, w2_ref, b2_ref, o_ref, *, mean_scale):
    xs = x_ref[...]

    pooled = jnp.sum(xs, axis=(0, 1), dtype=jnp.float32) * mean_scale

    hid = jax.lax.dot_general(
        pooled, w1t_ref[...], (((1,), (1,)), ((), ())),
        preferred_element_type=jnp.float32)
    hid = jnp.maximum(hid + b1_ref[...], 0.0)

    act = jax.lax.dot_general(
        hid, w2_ref[...], (((1,), (0,)), ((), ())),
        preferred_element_type=jnp.float32) + b2_ref[...]
    gate = jax.nn.sigmoid(act)

    o_ref[...] = xs * gate[None, None].astype(o_ref.dtype)


def _pick_batch_tile(B, per_batch_bytes):
    if B < 8 or B % 8:
        return B
    best = 8
    for cand in range(8, B + 1, 8):
        if B % cand:
            continue
        if cand * per_batch_bytes > 16 * 1024 * 1024 or B // cand < 4:
            break
        best = cand
    return best


@jax.jit
def kernel(x, w1, b1, w2, b2):
    B, C, H, W = x.shape
    Cr = w1.shape[1]

    xt = jnp.transpose(x, (2, 3, 0, 1))
    tile = _pick_batch_tile(B, C * H * W * x.dtype.itemsize)
    block_bytes = H * W * tile * C * x.dtype.itemsize
    vmem_limit = min(4 * block_bytes + 8 * 1024 * 1024, 112 * 1024 * 1024)

    body = functools.partial(_se_body, mean_scale=1.0 / (H * W))
    out_t = pl.pallas_call(
        body,
        out_shape=jax.ShapeDtypeStruct((H, W, B, C), x.dtype),
        grid=(B // tile,),
        in_specs=[
            pl.BlockSpec((H, W, tile, C), lambda b: (0, 0, b, 0)),
            pl.BlockSpec((Cr, C), lambda b: (0, 0)),
            pl.BlockSpec((1, Cr), lambda b: (0, 0)),
            pl.BlockSpec((Cr, C), lambda b: (0, 0)),
            pl.BlockSpec((1, C), lambda b: (0, 0)),
        ],
        out_specs=pl.BlockSpec((H, W, tile, C), lambda b: (0, 0, b, 0)),
        compiler_params=pltpu.CompilerParams(
            dimension_semantics=("parallel",),
            vmem_limit_bytes=vmem_limit),
    )(xt, w1.T, b1.reshape(1, Cr), w2, b2.reshape(1, C))

    return jnp.transpose(out_t, (2, 3, 0, 1))
